# baseline (speedup 1.0000x reference)
import jax
import jax.numpy as jnp
from jax import lax
from jax.experimental import pallas as pl
from jax.experimental.pallas import tpu as pltpu
from jax.experimental.pallas import tpu_sc as plsc

BATCH = 16384
EMB_DIM = 64
NC = 2
NS = 16
NW = NC * NS
B_PER_W = BATCH // NW
CHUNK = 128
N_CHUNKS = B_PER_W // CHUNK
LANES = 16


def _lookup_body(uid_idx_hbm, iid_idx_hbm, uid_table_hbm, iid_table_hbm,
                 uid_out_hbm, iid_out_hbm,
                 idx_v, row_v, off_v, rows2_a, rows2_b, out_a, out_b,
                 sem_g, sem_o):
    wid = lax.axis_index("s") * NC + lax.axis_index("c")
    base = wid * B_PER_W
    bufs = (rows2_a, rows2_b)
    obufs = (out_a, out_b)
    out_copies = []
    for table_hbm, src_idx_hbm, out_hbm in (
            (uid_table_hbm, uid_idx_hbm, uid_out_hbm),
            (iid_table_hbm, iid_idx_hbm, iid_out_hbm)):
        pltpu.sync_copy(src_idx_hbm.at[pl.ds(base, B_PER_W)], idx_v)
        for c in range(B_PER_W // LANES):
            sl = pl.ds(c * LANES, LANES)
            v = idx_v[sl]
            row_v[sl] = lax.shift_right_logical(v, 1)
            off_v[sl] = lax.shift_left(v & 1, 6)
        gathers = [pltpu.async_copy(
            table_hbm.at[row_v.at[pl.ds(0, CHUNK)]], bufs[0], sem_g)]
        for g in range(N_CHUNKS):
            if g + 1 < N_CHUNKS:
                gathers.append(pltpu.async_copy(
                    table_hbm.at[row_v.at[pl.ds((g + 1) * CHUNK, CHUNK)]],
                    bufs[(g + 1) % 2], sem_g))
            gathers[g].wait()
            if len(out_copies) >= 2:
                out_copies.pop(0).wait()
            buf = bufs[g % 2]
            out_v = obufs[g % 2]
            cbase = g * CHUNK

            def extract(r, carry, buf=buf, cbase=cbase, out_v=out_v):
                rowid = lax.iota(jnp.int32, LANES) + r * LANES
                srccol = off_v[pl.ds(cbase + r * LANES, LANES)]
                dstcol = jnp.zeros((LANES,), jnp.int32)
                one = jnp.ones((LANES,), jnp.int32)
                for _ in range(EMB_DIM):
                    vals = plsc.load_gather(buf, [rowid, srccol])
                    plsc.store_scatter(out_v, [rowid, dstcol], vals)
                    srccol = srccol + one
                    dstcol = dstcol + one
                return carry

            lax.fori_loop(0, CHUNK // LANES, extract, 0)
            out_copies.append(pltpu.async_copy(
                out_v, out_hbm.at[pl.ds(base + cbase, CHUNK)], sem_o))
    for c in out_copies:
        c.wait()


def kernel(x, uid_table, iid_table):
    uid_idx = x[:, 0]
    iid_idx = x[:, 1]
    tu = uid_table.reshape(UID_ROWS2, 2 * EMB_DIM)
    ti = iid_table.reshape(IID_ROWS2, 2 * EMB_DIM)
    mesh = plsc.VectorSubcoreMesh(core_axis_name="c", subcore_axis_name="s")
    f = pl.kernel(
        _lookup_body,
        out_type=(
            jax.ShapeDtypeStruct((BATCH, EMB_DIM), jnp.float32),
            jax.ShapeDtypeStruct((BATCH, EMB_DIM), jnp.float32),
        ),
        mesh=mesh,
        scratch_types=[
            pltpu.VMEM((B_PER_W,), jnp.int32),
            pltpu.VMEM((B_PER_W,), jnp.int32),
            pltpu.VMEM((B_PER_W,), jnp.int32),
            pltpu.VMEM((CHUNK, 2 * EMB_DIM), jnp.float32),
            pltpu.VMEM((CHUNK, 2 * EMB_DIM), jnp.float32),
            pltpu.VMEM((CHUNK, EMB_DIM), jnp.float32),
            pltpu.VMEM((CHUNK, EMB_DIM), jnp.float32),
            pltpu.SemaphoreType.DMA,
            pltpu.SemaphoreType.DMA,
        ],
        compiler_params=pltpu.CompilerParams(needs_layout_passes=False),
    )
    return f(uid_idx, iid_idx, tu, ti)


UID_ROWS2 = 500000
IID_ROWS2 = 500000


# speedup vs baseline: 1.6657x; 1.6657x over previous
import jax
import jax.numpy as jnp
from jax import lax
from jax.experimental import pallas as pl
from jax.experimental.pallas import tpu as pltpu
from jax.experimental.pallas import tpu_sc as plsc

BATCH = 16384
EMB_DIM = 64
NC = 2
NS = 16
NW = NC * NS
B_PER_W = BATCH // NW
CHUNK = 128
N_CHUNKS = B_PER_W // CHUNK
LANES = 16


def _lookup_body(uid_idx_hbm, iid_idx_hbm, uid_table_hbm, iid_table_hbm,
                 uid_out_hbm, iid_out_hbm,
                 idx_u, idx_i, rows_v, sem_g, sem_o):
    wid = lax.axis_index("s") * NC + lax.axis_index("c")
    base = wid * B_PER_W
    pltpu.sync_copy(uid_idx_hbm.at[pl.ds(base, B_PER_W)], idx_u)
    pltpu.sync_copy(iid_idx_hbm.at[pl.ds(base, B_PER_W)], idx_i)
    out_copy = None
    for table_hbm, idx_v, out_hbm in (
            (uid_table_hbm, idx_u, uid_out_hbm),
            (iid_table_hbm, idx_i, iid_out_hbm)):
        if out_copy is not None:
            out_copy.wait()

        def fetch(g, carry, table_hbm=table_hbm, idx_v=idx_v):
            vec = idx_v[pl.ds(g * LANES, LANES)]
            for l in range(LANES):
                pltpu.async_copy(table_hbm.at[pl.ds(vec[l], 1)],
                                 rows_v.at[pl.ds(g * LANES + l, 1)], sem_g)
            return carry

        lax.fori_loop(0, B_PER_W // LANES, fetch, 0)
        pltpu.make_async_copy(
            table_hbm.at[pl.ds(0, B_PER_W)], rows_v, sem_g).wait()
        out_copy = pltpu.async_copy(
            rows_v, out_hbm.at[pl.ds(base, B_PER_W)], sem_o)
    out_copy.wait()


def kernel(x, uid_table, iid_table):
    uid_idx = x[:, 0]
    iid_idx = x[:, 1]
    mesh = plsc.VectorSubcoreMesh(core_axis_name="c", subcore_axis_name="s")
    f = pl.kernel(
        _lookup_body,
        out_type=(
            jax.ShapeDtypeStruct((BATCH, EMB_DIM), jnp.float32),
            jax.ShapeDtypeStruct((BATCH, EMB_DIM), jnp.float32),
        ),
        mesh=mesh,
        scratch_types=[
            pltpu.VMEM((B_PER_W,), jnp.int32),
            pltpu.VMEM((B_PER_W,), jnp.int32),
            pltpu.VMEM((B_PER_W, EMB_DIM), jnp.float32),
            pltpu.SemaphoreType.DMA,
            pltpu.SemaphoreType.DMA,
        ],
        compiler_params=pltpu.CompilerParams(needs_layout_passes=False),
    )
    return f(uid_idx, iid_idx, uid_table, iid_table)
